# baseline (speedup 1.0000x reference)
import jax
import jax.numpy as jnp
from jax.experimental import pallas as pl
from jax.experimental.pallas import tpu as pltpu

_CHUNK = 16
_SUB = 8


def _gru_step(i, nb, hdim, h, m_ref, gi_scr, whh_ref, bhhn_ref):
    m_t = m_ref[i]
    hm = h * m_t
    hm_b = hm.astype(jnp.bfloat16)
    gi_t = gi_scr[i * nb:(i + 1) * nb, :]
    gh = jnp.dot(hm_b, whh_ref[...], preferred_element_type=jnp.float32)
    z = 0.5 * jnp.tanh(0.5 * (gi_t[:, hdim:2 * hdim]
                              + gh[:, hdim:2 * hdim])) + 0.5
    r = 0.5 * jnp.tanh(0.5 * (gi_t[:, :hdim] + gh[:, :hdim])) + 0.5
    n = jnp.tanh(gi_t[:, 2 * hdim:]
                 + r * (gh[:, 2 * hdim:] + bhhn_ref[...]))
    return n * (1.0 - z) + z * hm


def _fused_kernel(x_ref, m0_ref, m1_ref, h0_ref, h1_ref,
                  wih0_ref, bi0_ref, whh0_ref, bn0_ref,
                  wih1_ref, bi1_ref, whh1_ref, bn1_ref,
                  out_ref, h0n_ref, h1n_ref,
                  h0_scr, h1_scr, gi0_scr, gi1_scr, out0_scr):
    c = pl.program_id(0)
    ngrid = pl.num_programs(0)
    hdim = h0_ref.shape[-1]
    chunk = m0_ref.shape[0]
    nb = h0_ref.shape[0]
    sub = _SUB
    nsub = chunk // sub

    @pl.when(c == 0)
    def _():
        h0_scr[...] = h0_ref[...]
        h1_scr[...] = h1_ref[...]

    gi0_scr[...] = jnp.dot(x_ref[...].astype(jnp.bfloat16), wih0_ref[...],
                           preferred_element_type=jnp.float32) + bi0_ref[...]

    a = h0_scr[...]
    b = h1_scr[...]

    for k in range(nsub):
        for i in range(sub):
            i0 = k * sub + i
            b = _gru_step(i0, nb, hdim, b, m1_ref, gi1_scr,
                          whh1_ref, bn1_ref)
            out_ref[i0 * nb:(i0 + 1) * nb, :] = b
            a = _gru_step(i0, nb, hdim, a, m0_ref, gi0_scr,
                          whh0_ref, bn0_ref)
            out0_scr[i0 * nb:(i0 + 1) * nb, :] = a.astype(jnp.bfloat16)
        r0, r1 = k * sub * nb, (k + 1) * sub * nb
        gi1_scr[r0:r1, :] = jnp.dot(
            out0_scr[r0:r1, :], wih1_ref[...],
            preferred_element_type=jnp.float32) + bi1_ref[...]

    @pl.when(c < ngrid - 1)
    def _():
        h0_scr[...] = a

    @pl.when(c > 0)
    def _():
        h1_scr[...] = b

    @pl.when(c == ngrid - 2)
    def _():
        h0n_ref[...] = a

    @pl.when(c == ngrid - 1)
    def _():
        h1n_ref[...] = b


def kernel(x, hidden_states, masks, W_ih_0, W_hh_0, b_ih_0, b_hh_0,
           W_ih_1, W_hh_1, b_ih_1, b_hh_1):
    n = hidden_states.shape[1]
    tn, d = x.shape
    t = tn // n
    h = hidden_states.shape[2]
    g3 = 3 * h
    chunk = _CHUNK
    rows = chunk * n
    nchunks = t // chunk

    m3 = masks.reshape(t, n, 1).astype(jnp.float32)

    def gi_bias(b_ih, b_hh):
        return jnp.concatenate(
            [b_ih[:2 * h] + b_hh[:2 * h], b_ih[2 * h:]]).reshape(1, g3)

    bi0 = gi_bias(b_ih_0, b_hh_0)
    bi1 = gi_bias(b_ih_1, b_hh_1)
    bn0 = b_hh_0[2 * h:].reshape(1, h)
    bn1 = b_hh_1[2 * h:].reshape(1, h)

    def lead(c):
        return jnp.minimum(c, nchunks - 1)

    def lag(c):
        return jnp.maximum(c - 1, 0)

    call = pl.pallas_call(
        _fused_kernel,
        grid=(nchunks + 1,),
        in_specs=[
            pl.BlockSpec((rows, d), lambda c: (lead(c), 0)),
            pl.BlockSpec((chunk, n, 1), lambda c: (lead(c), 0, 0)),
            pl.BlockSpec((chunk, n, 1), lambda c: (lag(c), 0, 0)),
            pl.BlockSpec((n, h), lambda c: (0, 0)),
            pl.BlockSpec((n, h), lambda c: (0, 0)),
            pl.BlockSpec((d, g3), lambda c: (0, 0)),
            pl.BlockSpec((1, g3), lambda c: (0, 0)),
            pl.BlockSpec((h, g3), lambda c: (0, 0)),
            pl.BlockSpec((1, h), lambda c: (0, 0)),
            pl.BlockSpec((h, g3), lambda c: (0, 0)),
            pl.BlockSpec((1, g3), lambda c: (0, 0)),
            pl.BlockSpec((h, g3), lambda c: (0, 0)),
            pl.BlockSpec((1, h), lambda c: (0, 0)),
        ],
        out_specs=[
            pl.BlockSpec((rows, h), lambda c: (lag(c), 0)),
            pl.BlockSpec((n, h), lambda c: (0, 0)),
            pl.BlockSpec((n, h), lambda c: (0, 0)),
        ],
        out_shape=[
            jax.ShapeDtypeStruct((tn, h), jnp.float32),
            jax.ShapeDtypeStruct((n, h), jnp.float32),
            jax.ShapeDtypeStruct((n, h), jnp.float32),
        ],
        scratch_shapes=[
            pltpu.VMEM((n, h), jnp.float32),
            pltpu.VMEM((n, h), jnp.float32),
            pltpu.VMEM((rows, g3), jnp.float32),
            pltpu.VMEM((rows, g3), jnp.float32),
            pltpu.VMEM((rows, h), jnp.bfloat16),
        ],
        compiler_params=pltpu.CompilerParams(
            dimension_semantics=("arbitrary",),
        ),
    )

    args = (x, m3, m3, hidden_states[0], hidden_states[1],
            W_ih_0.T.astype(jnp.bfloat16), bi0,
            W_hh_0.T.astype(jnp.bfloat16), bn0,
            W_ih_1.T.astype(jnp.bfloat16), bi1,
            W_hh_1.T.astype(jnp.bfloat16), bn1)

    out, h0n, h1n = call(*args)
    return out, jnp.stack([h0n, h1n], axis=0)


# speedup vs baseline: 1.0213x; 1.0213x over previous
import jax
import jax.numpy as jnp
from jax.experimental import pallas as pl
from jax.experimental.pallas import tpu as pltpu

_CHUNK = 16
_SUB = 16


def _gru_step(i, nb, hdim, h, m_ref, gi_scr, whh_ref, bhhn_ref):
    m_t = m_ref[i]
    hm = h * m_t
    hm_b = hm.astype(jnp.bfloat16)
    gi_t = gi_scr[i * nb:(i + 1) * nb, :]
    gh = jnp.dot(hm_b, whh_ref[...], preferred_element_type=jnp.float32)
    z = 0.5 * jnp.tanh(0.5 * (gi_t[:, hdim:2 * hdim]
                              + gh[:, hdim:2 * hdim])) + 0.5
    r = 0.5 * jnp.tanh(0.5 * (gi_t[:, :hdim] + gh[:, :hdim])) + 0.5
    n = jnp.tanh(gi_t[:, 2 * hdim:]
                 + r * (gh[:, 2 * hdim:] + bhhn_ref[...]))
    return n * (1.0 - z) + z * hm


def _fused_kernel(x_ref, m0_ref, m1_ref, h0_ref, h1_ref,
                  wih0_ref, bi0_ref, whh0_ref, bn0_ref,
                  wih1_ref, bi1_ref, whh1_ref, bn1_ref,
                  out_ref, h0n_ref, h1n_ref,
                  h0_scr, h1_scr, gi0_scr, gi1_scr, out0_scr):
    c = pl.program_id(0)
    ngrid = pl.num_programs(0)
    hdim = h0_ref.shape[-1]
    chunk = m0_ref.shape[0]
    nb = h0_ref.shape[0]
    sub = _SUB
    nsub = chunk // sub

    @pl.when(c == 0)
    def _():
        h0_scr[...] = h0_ref[...]
        h1_scr[...] = h1_ref[...]

    gi0_scr[...] = jnp.dot(x_ref[...].astype(jnp.bfloat16), wih0_ref[...],
                           preferred_element_type=jnp.float32) + bi0_ref[...]

    a = h0_scr[...]
    b = h1_scr[...]

    for k in range(nsub):
        for i in range(sub):
            i0 = k * sub + i
            b = _gru_step(i0, nb, hdim, b, m1_ref, gi1_scr,
                          whh1_ref, bn1_ref)
            out_ref[i0 * nb:(i0 + 1) * nb, :] = b
            a = _gru_step(i0, nb, hdim, a, m0_ref, gi0_scr,
                          whh0_ref, bn0_ref)
            out0_scr[i0 * nb:(i0 + 1) * nb, :] = a.astype(jnp.bfloat16)
        r0, r1 = k * sub * nb, (k + 1) * sub * nb
        gi1_scr[r0:r1, :] = jnp.dot(
            out0_scr[r0:r1, :], wih1_ref[...],
            preferred_element_type=jnp.float32) + bi1_ref[...]

    @pl.when(c < ngrid - 1)
    def _():
        h0_scr[...] = a

    @pl.when(c > 0)
    def _():
        h1_scr[...] = b

    @pl.when(c == ngrid - 2)
    def _():
        h0n_ref[...] = a

    @pl.when(c == ngrid - 1)
    def _():
        h1n_ref[...] = b


def kernel(x, hidden_states, masks, W_ih_0, W_hh_0, b_ih_0, b_hh_0,
           W_ih_1, W_hh_1, b_ih_1, b_hh_1):
    n = hidden_states.shape[1]
    tn, d = x.shape
    t = tn // n
    h = hidden_states.shape[2]
    g3 = 3 * h
    chunk = _CHUNK
    rows = chunk * n
    nchunks = t // chunk

    m3 = masks.reshape(t, n, 1).astype(jnp.float32)

    def gi_bias(b_ih, b_hh):
        return jnp.concatenate(
            [b_ih[:2 * h] + b_hh[:2 * h], b_ih[2 * h:]]).reshape(1, g3)

    bi0 = gi_bias(b_ih_0, b_hh_0)
    bi1 = gi_bias(b_ih_1, b_hh_1)
    bn0 = b_hh_0[2 * h:].reshape(1, h)
    bn1 = b_hh_1[2 * h:].reshape(1, h)

    def lead(c):
        return jnp.minimum(c, nchunks - 1)

    def lag(c):
        return jnp.maximum(c - 1, 0)

    call = pl.pallas_call(
        _fused_kernel,
        grid=(nchunks + 1,),
        in_specs=[
            pl.BlockSpec((rows, d), lambda c: (lead(c), 0)),
            pl.BlockSpec((chunk, n, 1), lambda c: (lead(c), 0, 0)),
            pl.BlockSpec((chunk, n, 1), lambda c: (lag(c), 0, 0)),
            pl.BlockSpec((n, h), lambda c: (0, 0)),
            pl.BlockSpec((n, h), lambda c: (0, 0)),
            pl.BlockSpec((d, g3), lambda c: (0, 0)),
            pl.BlockSpec((1, g3), lambda c: (0, 0)),
            pl.BlockSpec((h, g3), lambda c: (0, 0)),
            pl.BlockSpec((1, h), lambda c: (0, 0)),
            pl.BlockSpec((h, g3), lambda c: (0, 0)),
            pl.BlockSpec((1, g3), lambda c: (0, 0)),
            pl.BlockSpec((h, g3), lambda c: (0, 0)),
            pl.BlockSpec((1, h), lambda c: (0, 0)),
        ],
        out_specs=[
            pl.BlockSpec((rows, h), lambda c: (lag(c), 0)),
            pl.BlockSpec((n, h), lambda c: (0, 0)),
            pl.BlockSpec((n, h), lambda c: (0, 0)),
        ],
        out_shape=[
            jax.ShapeDtypeStruct((tn, h), jnp.float32),
            jax.ShapeDtypeStruct((n, h), jnp.float32),
            jax.ShapeDtypeStruct((n, h), jnp.float32),
        ],
        scratch_shapes=[
            pltpu.VMEM((n, h), jnp.float32),
            pltpu.VMEM((n, h), jnp.float32),
            pltpu.VMEM((rows, g3), jnp.float32),
            pltpu.VMEM((rows, g3), jnp.float32),
            pltpu.VMEM((rows, h), jnp.bfloat16),
        ],
        compiler_params=pltpu.CompilerParams(
            dimension_semantics=("arbitrary",),
        ),
    )

    args = (x, m3, m3, hidden_states[0], hidden_states[1],
            W_ih_0.T.astype(jnp.bfloat16), bi0,
            W_hh_0.T.astype(jnp.bfloat16), bn0,
            W_ih_1.T.astype(jnp.bfloat16), bi1,
            W_hh_1.T.astype(jnp.bfloat16), bn1)

    out, h0n, h1n = call(*args)
    return out, jnp.stack([h0n, h1n], axis=0)
